# Initial kernel scaffold; baseline (speedup 1.0000x reference)
import jax
import jax.numpy as jnp
from jax.experimental import pallas as pl


def kernel(Z_hat, B, w, agent_idx):
    raise NotImplementedError("write your pallas kernel here")


# speedup vs baseline: 1.0381x; 1.0381x over previous
import functools

import jax
import jax.numpy as jnp
from jax.experimental import pallas as pl
from jax.experimental.pallas import tpu as pltpu

NUM_AGENTS = 8
HIDDEN = 1024
NZ = 2048
BATCH = 16384


def _scale_kernel(aidx_ref, b_ref, w_ref, out_ref, masks_ref):
    a = pl.program_id(0)
    m = (jnp.max(b_ref[0], axis=0) > 0).astype(jnp.float32)
    masks_ref[a, :] = m

    @pl.when(a == NUM_AGENTS - 1)
    def _finalize():
        alpha = jnp.sum(masks_ref[...], axis=0)
        aidx = aidx_ref[0]
        mask_sel = masks_ref[pl.ds(aidx, 1), :][0]
        weights = jnp.zeros((NZ,), jnp.float32)
        for k in range(NUM_AGENTS + 1):
            weights = jnp.where(alpha == float(k), w_ref[0, k], weights)
        out_ref[0, :] = mask_sel * weights


def _mul_kernel(z_ref, s_ref, out_ref):
    out_ref[...] = z_ref[...] * s_ref[...]


@functools.partial(jax.jit, static_argnames=())
def kernel(Z_hat, B, w, agent_idx):
    B3 = B.reshape(NUM_AGENTS, HIDDEN, NZ)
    w2 = jnp.zeros((1, 16), jnp.float32).at[0, : NUM_AGENTS + 1].set(w)
    aidx = jnp.asarray(agent_idx, jnp.int32).reshape((1,))

    scale = pl.pallas_call(
        _scale_kernel,
        grid_spec=pltpu.PrefetchScalarGridSpec(
            num_scalar_prefetch=1,
            grid=(NUM_AGENTS,),
            in_specs=[
                pl.BlockSpec((1, HIDDEN, NZ), lambda a, aidx: (a, 0, 0)),
                pl.BlockSpec((1, 16), lambda a, aidx: (0, 0)),
            ],
            out_specs=pl.BlockSpec((1, NZ), lambda a, aidx: (0, 0)),
            scratch_shapes=[pltpu.VMEM((NUM_AGENTS, NZ), jnp.float32)],
        ),
        out_shape=jax.ShapeDtypeStruct((1, NZ), jnp.float32),
    )(aidx, B3, w2)

    ROWS = 1024
    out = pl.pallas_call(
        _mul_kernel,
        grid=(BATCH // ROWS,),
        in_specs=[
            pl.BlockSpec((ROWS, NZ), lambda i: (i, 0)),
            pl.BlockSpec((1, NZ), lambda i: (0, 0)),
        ],
        out_specs=pl.BlockSpec((ROWS, NZ), lambda i: (i, 0)),
        out_shape=jax.ShapeDtypeStruct((BATCH, NZ), jnp.float32),
    )(Z_hat, scale)
    return out
